# Initial kernel scaffold; baseline (speedup 1.0000x reference)
import jax
import jax.numpy as jnp
from jax.experimental import pallas as pl


def kernel(features, edge_index, edge_weight, W1, b1, W2, b2):
    raise NotImplementedError("write your pallas kernel here")


# speedup vs baseline: 10.2745x; 10.2745x over previous
import functools

import jax
import jax.numpy as jnp
from jax import lax
from jax.experimental import pallas as pl
from jax.experimental.pallas import tpu as pltpu
from jax.experimental.pallas import tpu_sc as plsc

NUM_CORES = 2
NUM_SUBCORES = 16
NW = NUM_CORES * NUM_SUBCORES
LANES = 16
B = 16
DEPTH = 4


def _spmm_sc(y, src3, dst3, w3, zeros, n_pad, d):
  nchunk = src3.shape[1]
  rows_per_sub = n_pad // NUM_SUBCORES
  mesh = plsc.VectorSubcoreMesh(
      core_axis_name="c", subcore_axis_name="s",
      num_cores=NUM_CORES, num_subcores=NUM_SUBCORES)

  @functools.partial(
      pl.kernel,
      out_type=jax.ShapeDtypeStruct((NUM_CORES, n_pad, d), jnp.float32),
      mesh=mesh,
      scratch_types=[
          pltpu.VMEM((nchunk, B), jnp.int32),
          pltpu.VMEM((nchunk, B), jnp.int32),
          pltpu.VMEM((nchunk * B,), jnp.float32),
          pltpu.VMEM((DEPTH, B, d), jnp.float32),
          pltpu.VMEM_SHARED((n_pad, d), jnp.float32),
          [pltpu.SemaphoreType.DMA] * DEPTH,
      ],
      compiler_params=pltpu.CompilerParams(use_tc_tiling_on_sc=False),
  )
  def spmm(y_hbm, src_hbm, dst_hbm, w_hbm, zeros_hbm, out_hbm,
           src_v, dst_v, w_v, rows_v, acc, gsems):
    cid = lax.axis_index("c")
    sid = lax.axis_index("s")
    wid = cid * NUM_SUBCORES + sid

    pltpu.sync_copy(src_hbm.at[wid], src_v)
    pltpu.sync_copy(dst_hbm.at[wid], dst_v)
    pltpu.sync_copy(w_hbm.at[wid], w_v)
    pltpu.sync_copy(zeros_hbm,
                    acc.at[pl.ds(sid * rows_per_sub, rows_per_sub)])
    plsc.subcore_barrier()

    for par in range(DEPTH):
      pltpu.async_copy(y_hbm.at[src_v.at[par]], rows_v.at[par], gsems[par])

    @pl.loop(0, nchunk, step=DEPTH)
    def _round(c0):
      for par in range(DEPTH):
        c = c0 + par

        @pl.when(c < nchunk)
        def _chunk():
          pltpu.make_async_copy(
              y_hbm.at[src_v.at[c]], rows_v.at[par], gsems[par]).wait()

          w16 = w_v[pl.ds(c * B, LANES)]
          for j in range(B):
            wj = jnp.take_along_axis(
                w16, jnp.full((LANES,), j, jnp.int32), axis=0)
            for k in range(d // LANES):
              sl = pl.ds(k * LANES, LANES)
              rows_v[par, j, sl] = rows_v[par, j, sl] * wj

          pltpu.sync_copy(rows_v.at[par], acc.at[dst_v.at[c]], add=True)

          @pl.when(c + DEPTH < nchunk)
          def _refill():
            pltpu.async_copy(
                y_hbm.at[src_v.at[c + DEPTH]], rows_v.at[par], gsems[par])

    plsc.subcore_barrier()
    pltpu.sync_copy(
        acc.at[pl.ds(sid * rows_per_sub, rows_per_sub)],
        out_hbm.at[cid, pl.ds(sid * rows_per_sub, rows_per_sub)])

  return spmm(y, src3, dst3, w3, zeros)


def _mm_tc(x, w):
  rows = x.shape[0]
  blk = 2000 if rows % 2000 == 0 else rows
  grid = rows // blk

  def body(x_ref, w_ref, o_ref):
    o_ref[...] = jnp.dot(x_ref[...], w_ref[...],
                         preferred_element_type=jnp.float32)

  return pl.pallas_call(
      body,
      grid=(grid,),
      in_specs=[
          pl.BlockSpec((blk, x.shape[1]), lambda i: (i, 0)),
          pl.BlockSpec(w.shape, lambda i: (0, 0)),
      ],
      out_specs=pl.BlockSpec((blk, w.shape[1]), lambda i: (i, 0)),
      out_shape=jax.ShapeDtypeStruct((rows, w.shape[1]), jnp.float32),
  )(x, w)


def _combine_relu_mm_tc(p, b1, w2):
  rows, k = p.shape[1], p.shape[2]
  blk = 2048
  grid = rows // blk

  def body(p_ref, b_ref, w_ref, o_ref):
    h = jnp.maximum(p_ref[0] + p_ref[1] + b_ref[...], 0.0)
    o_ref[...] = jnp.dot(h, w_ref[...], preferred_element_type=jnp.float32)

  return pl.pallas_call(
      body,
      grid=(grid,),
      in_specs=[
          pl.BlockSpec((2, blk, k), lambda i: (0, i, 0)),
          pl.BlockSpec((1, k), lambda i: (0, 0)),
          pl.BlockSpec(w2.shape, lambda i: (0, 0)),
      ],
      out_specs=pl.BlockSpec((blk, w2.shape[1]), lambda i: (i, 0)),
      out_shape=jax.ShapeDtypeStruct((rows, w2.shape[1]), jnp.float32),
  )(p, b1, w2)


def _final_tc(p, b2):
  rows, c = p.shape[1], p.shape[2]

  def body(p_ref, b_ref, o_ref):
    o_ref[...] = p_ref[0] + p_ref[1] + b_ref[...]

  return pl.pallas_call(
      body,
      grid=(1,),
      in_specs=[
          pl.BlockSpec((2, rows, c), lambda i: (0, 0, 0)),
          pl.BlockSpec((1, c), lambda i: (0, 0)),
      ],
      out_specs=pl.BlockSpec((rows, c), lambda i: (0, 0)),
      out_shape=jax.ShapeDtypeStruct((rows, c), jnp.float32),
  )(p, b2)


def kernel(features, edge_index, edge_weight, W1, b1, W2, b2):
  n, dd = features.shape
  h = W1.shape[1]
  c = W2.shape[1]
  e = edge_weight.shape[0]

  epw = e // NW
  nchunk = epw // B
  assert epw * NW == e and nchunk * B == epw

  n_pad = ((n + NW * 8 - 1) // (NW * 8)) * (NW * 8)
  rows_per_sub = n_pad // NUM_SUBCORES

  src3 = edge_index[0].reshape(NW, nchunk, B)
  dst3 = edge_index[1].reshape(NW, nchunk, B)
  w3 = edge_weight.reshape(NW, nchunk * B)
  zeros_h = jnp.zeros((rows_per_sub, h), jnp.float32)
  zeros_c = jnp.zeros((rows_per_sub, c), jnp.float32)

  y1 = _mm_tc(features, W1)
  p1 = _spmm_sc(y1, src3, dst3, w3, zeros_h, n_pad, h)
  y2 = _combine_relu_mm_tc(p1, b1.reshape(1, h), W2)
  p2 = _spmm_sc(y2, src3, dst3, w3, zeros_c, n_pad, c)
  out = _final_tc(p2, b2.reshape(1, c))
  return out[:n]


# speedup vs baseline: 15.0400x; 1.4638x over previous
import functools

import jax
import jax.numpy as jnp
from jax import lax
from jax.experimental import pallas as pl
from jax.experimental.pallas import tpu as pltpu
from jax.experimental.pallas import tpu_sc as plsc

NUM_CORES = 2
NUM_SUBCORES = 16
NW = NUM_CORES * NUM_SUBCORES
LANES = 16
B = 80
DEPTH = 3


def _spmm_sc(y, src3, dst3, w3, zeros, n_pad, d):
  nchunk = src3.shape[1]
  rows_per_sub = n_pad // NUM_SUBCORES
  mesh = plsc.VectorSubcoreMesh(
      core_axis_name="c", subcore_axis_name="s",
      num_cores=NUM_CORES, num_subcores=NUM_SUBCORES)

  @functools.partial(
      pl.kernel,
      out_type=jax.ShapeDtypeStruct((NUM_CORES, n_pad, d), jnp.float32),
      mesh=mesh,
      scratch_types=[
          pltpu.VMEM((nchunk, B), jnp.int32),
          pltpu.VMEM((DEPTH, B), jnp.int32),
          pltpu.VMEM((DEPTH, B), jnp.float32),
          pltpu.VMEM((DEPTH, B, d), jnp.float32),
          pltpu.VMEM_SHARED((n_pad, d), jnp.float32),
          [pltpu.SemaphoreType.DMA] * DEPTH,
          [pltpu.SemaphoreType.DMA] * DEPTH,
      ],
      compiler_params=pltpu.CompilerParams(use_tc_tiling_on_sc=False),
  )
  def spmm(y_hbm, src_hbm, dst_hbm, w_hbm, zeros_hbm, out_hbm,
           src_v, dst_v, w_v, rows_v, acc, gsems, esems):
    cid = lax.axis_index("c")
    sid = lax.axis_index("s")
    wid = cid * NUM_SUBCORES + sid

    pltpu.sync_copy(src_hbm.at[wid], src_v)
    pltpu.sync_copy(zeros_hbm,
                    acc.at[pl.ds(sid * rows_per_sub, rows_per_sub)])
    plsc.subcore_barrier()

    def issue(c, par):
      pltpu.async_copy(y_hbm.at[src_v.at[c]], rows_v.at[par], gsems[par])
      pltpu.async_copy(dst_hbm.at[wid, c], dst_v.at[par], esems[par])
      pltpu.async_copy(w_hbm.at[wid, c], w_v.at[par], esems[par])

    def wait(c, par):
      pltpu.make_async_copy(
          y_hbm.at[src_v.at[c]], rows_v.at[par], gsems[par]).wait()
      pltpu.make_async_copy(
          dst_hbm.at[wid, c], dst_v.at[par], esems[par]).wait()
      pltpu.make_async_copy(
          w_hbm.at[wid, c], w_v.at[par], esems[par]).wait()

    for par in range(DEPTH):
      issue(par, par)

    @pl.loop(0, nchunk, step=DEPTH)
    def _round(c0):
      for par in range(DEPTH):
        c = c0 + par

        @pl.when(c < nchunk)
        def _chunk():
          wait(c, par)

          @pl.loop(0, B // LANES)
          def _grp(g):
            w16 = w_v[par, pl.ds(g * LANES, LANES)]
            for j in range(LANES):
              wj = jnp.take_along_axis(
                  w16, jnp.full((LANES,), j, jnp.int32), axis=0)
              for k in range(d // LANES):
                sl = pl.ds(k * LANES, LANES)
                rows_v[par, g * LANES + j, sl] = (
                    rows_v[par, g * LANES + j, sl] * wj)

          pltpu.sync_copy(rows_v.at[par], acc.at[dst_v.at[par]], add=True)

          @pl.when(c + DEPTH < nchunk)
          def _refill():
            issue(c + DEPTH, par)

    plsc.subcore_barrier()
    pltpu.sync_copy(
        acc.at[pl.ds(sid * rows_per_sub, rows_per_sub)],
        out_hbm.at[cid, pl.ds(sid * rows_per_sub, rows_per_sub)])

  return spmm(y, src3, dst3, w3, zeros)


def _mm_tc(x, w):
  rows = x.shape[0]
  blk = 2000 if rows % 2000 == 0 else rows
  grid = rows // blk

  def body(x_ref, w_ref, o_ref):
    o_ref[...] = jnp.dot(x_ref[...], w_ref[...],
                         preferred_element_type=jnp.float32)

  return pl.pallas_call(
      body,
      grid=(grid,),
      in_specs=[
          pl.BlockSpec((blk, x.shape[1]), lambda i: (i, 0)),
          pl.BlockSpec(w.shape, lambda i: (0, 0)),
      ],
      out_specs=pl.BlockSpec((blk, w.shape[1]), lambda i: (i, 0)),
      out_shape=jax.ShapeDtypeStruct((rows, w.shape[1]), jnp.float32),
  )(x, w)


def _combine_relu_mm_tc(p, b1, w2):
  rows, k = p.shape[1], p.shape[2]
  blk = 2048
  grid = rows // blk

  def body(p_ref, b_ref, w_ref, o_ref):
    h = jnp.maximum(p_ref[0] + p_ref[1] + b_ref[...], 0.0)
    o_ref[...] = jnp.dot(h, w_ref[...], preferred_element_type=jnp.float32)

  return pl.pallas_call(
      body,
      grid=(grid,),
      in_specs=[
          pl.BlockSpec((2, blk, k), lambda i: (0, i, 0)),
          pl.BlockSpec((1, k), lambda i: (0, 0)),
          pl.BlockSpec(w2.shape, lambda i: (0, 0)),
      ],
      out_specs=pl.BlockSpec((blk, w2.shape[1]), lambda i: (i, 0)),
      out_shape=jax.ShapeDtypeStruct((rows, w2.shape[1]), jnp.float32),
  )(p, b1, w2)


def _final_tc(p, b2):
  rows, c = p.shape[1], p.shape[2]

  def body(p_ref, b_ref, o_ref):
    o_ref[...] = p_ref[0] + p_ref[1] + b_ref[...]

  return pl.pallas_call(
      body,
      grid=(1,),
      in_specs=[
          pl.BlockSpec((2, rows, c), lambda i: (0, 0, 0)),
          pl.BlockSpec((1, c), lambda i: (0, 0)),
      ],
      out_specs=pl.BlockSpec((rows, c), lambda i: (0, 0)),
      out_shape=jax.ShapeDtypeStruct((rows, c), jnp.float32),
  )(p, b2)


def kernel(features, edge_index, edge_weight, W1, b1, W2, b2):
  n, dd = features.shape
  h = W1.shape[1]
  c = W2.shape[1]
  e = edge_weight.shape[0]

  epw = e // NW
  nchunk = epw // B
  assert epw * NW == e and nchunk * B == epw

  n_pad = ((n + NW * 8 - 1) // (NW * 8)) * (NW * 8)
  rows_per_sub = n_pad // NUM_SUBCORES

  src3 = edge_index[0].reshape(NW, nchunk, B)
  dst3 = edge_index[1].reshape(NW, nchunk, B)
  w3 = edge_weight.reshape(NW, nchunk, B)
  zeros_h = jnp.zeros((rows_per_sub, h), jnp.float32)
  zeros_c = jnp.zeros((rows_per_sub, c), jnp.float32)

  y1 = _mm_tc(features, W1)
  p1 = _spmm_sc(y1, src3, dst3, w3, zeros_h, n_pad, h)
  y2 = _combine_relu_mm_tc(p1, b1.reshape(1, h), W2)
  p2 = _spmm_sc(y2, src3, dst3, w3, zeros_c, n_pad, c)
  out = _final_tc(p2, b2.reshape(1, c))
  return out[:n]
